# baseline (speedup 1.0000x reference)
import functools

import jax
import jax.numpy as jnp
from jax import lax
from jax.experimental import pallas as pl
from jax.experimental.pallas import tpu as pltpu
from jax.experimental.pallas import tpu_sc as plsc

N = 10000
NP = 10240
E = 320000
D = 128
DE = 16
NW = 32

EPW = E // NW
GC = 400
GN = EPW // GC

_sc_mesh = plsc.VectorSubcoreMesh(core_axis_name="c", subcore_axis_name="s")


@functools.partial(
    pl.kernel,
    mesh=_sc_mesh,
    out_type=jax.ShapeDtypeStruct((E, D), jnp.float32),
    scratch_types=[
        pltpu.VMEM((EPW,), jnp.int32),
        pltpu.VMEM((GC, D), jnp.float32),
        pltpu.VMEM((GC, D), jnp.float32),
        pltpu.SemaphoreType.DMA,
        pltpu.SemaphoreType.DMA,
    ],
)
def _sc_gather(table_hbm, idx_hbm, out_hbm, idxa, rows0, rows1, sg0, sg1):
    wid = lax.axis_index("s") * 2 + lax.axis_index("c")
    base = pl.multiple_of(wid * EPW, 8)
    rows = (rows0, rows1)
    sgs = (sg0, sg1)

    pltpu.sync_copy(idx_hbm.at[pl.ds(base, EPW)], idxa)

    def fire(ci, b):
        o = pl.multiple_of(ci * GC, 8)
        pltpu.async_copy(table_hbm.at[idxa.at[pl.ds(o, GC)]], rows[b], sgs[b])

    def process(ci, b):
        o = pl.multiple_of(ci * GC, 8)
        pltpu.make_async_copy(table_hbm.at[idxa.at[pl.ds(o, GC)]],
                              rows[b], sgs[b]).wait()
        off = pl.multiple_of(base + ci * GC, 8)
        pltpu.sync_copy(rows[b], out_hbm.at[pl.ds(off, GC)])

    fire(0, 0)
    fire(1, 1)

    def pair(k, _):
        ci = k * 2
        process(ci, 0)

        @pl.when(ci + 2 < GN)
        def _f0():
            fire(ci + 2, 0)
        process(ci + 1, 1)

        @pl.when(ci + 3 < GN)
        def _f1():
            fire(ci + 3, 1)
        return _

    lax.fori_loop(0, GN // 2, pair, 0)
    if GN % 2:
        process(GN - 1, 0)


RPW = 8
EH = E // 2
SK = 1280
SN = EH // SK


@functools.partial(
    pl.kernel,
    mesh=_sc_mesh,
    out_type=jax.ShapeDtypeStruct((2, D, NP), jnp.float32),
    scratch_types=[
        [pltpu.VMEM((NP,), jnp.float32) for _ in range(RPW)],
        pltpu.VMEM((SK,), jnp.int32),
        pltpu.VMEM((SK,), jnp.int32),
        pltpu.VMEM((RPW, SK), jnp.float32),
        pltpu.VMEM((RPW, SK), jnp.float32),
        pltpu.VMEM((NP,), jnp.int32),
        pltpu.SemaphoreType.DMA,
        pltpu.SemaphoreType.DMA,
        pltpu.SemaphoreType.DMA,
        pltpu.SemaphoreType.DMA,
    ],
    compiler_params=pltpu.CompilerParams(needs_layout_passes=False),
)
def _sc_segmax(msgT_hbm, dst_hbm, out_hbm, accs, dstc0, dstc1, mv0, mv1,
               tag, sd0, sd1, sm0, sm1):
    wid = lax.axis_index("s") * 2 + lax.axis_index("c")
    g = wid % 16
    h = wid // 16
    r0 = pl.multiple_of(g * RPW, 8)
    eb = pl.multiple_of(h * EH, 128)
    zeros = jnp.zeros((16,), jnp.float32)
    dstcs = (dstc0, dstc1)
    mvs = (mv0, mv1)
    sds = (sd0, sd1)
    sms = (sm0, sm1)

    def zbody(j, _):
        for c in range(RPW):
            accs[c][pl.ds(j * 16, 16)] = zeros
        return _
    lax.fori_loop(0, NP // 16, zbody, 0)

    def _srcs(ci):
        e0 = pl.multiple_of(eb + ci * SK, 128)
        return (dst_hbm.at[pl.ds(e0, SK)],
                msgT_hbm.at[pl.ds(r0, RPW), pl.ds(e0, SK)])

    def fire(ci, b):
        sd, sm = _srcs(ci)
        pltpu.async_copy(sd, dstcs[b], sds[b])
        pltpu.async_copy(sm, mvs[b], sms[b])

    def process(ci, b):
        dstc, mv = dstcs[b], mvs[b]
        sd, sm = _srcs(ci)
        pltpu.make_async_copy(sd, dstc, sds[b]).wait()
        pltpu.make_async_copy(sm, mv, sms[b]).wait()

        lanes = lax.iota(jnp.int32, 16)

        NG = 8

        def octgrp(jp, _):
            j0 = jp * NG
            anydup = None
            for q in range(NG):
                dv = dstc[pl.ds((j0 + q) * 16, 16)]
                plsc.store_scatter(tag, [dv], lanes)
                rb = plsc.load_gather(tag, [dv])
                d = rb != lanes
                anydup = d if anydup is None else (anydup | d)

            for q in range(NG):
                dv = dstc[pl.ds((j0 + q) * 16, 16)]
                for c in range(RPW):
                    v = mv[c, pl.ds((j0 + q) * 16, 16)]
                    a = plsc.load_gather(accs[c], [dv])
                    plsc.store_scatter(accs[c], [dv], jnp.maximum(a, v))

            @pl.when(jnp.any(anydup))
            def _verify():
                for q in range(NG):
                    dv = dstc[pl.ds((j0 + q) * 16, 16)]
                    pendor = None
                    for c in range(RPW):
                        v = mv[c, pl.ds((j0 + q) * 16, 16)]
                        gt = plsc.load_gather(accs[c], [dv])
                        p = gt < v
                        pendor = p if pendor is None else (pendor | p)

                    @pl.when(jnp.any(pendor))
                    def _fix(q=q, dv=dv):
                        for c in range(RPW):
                            v = mv[c, pl.ds((j0 + q) * 16, 16)]
                            gt = plsc.load_gather(accs[c], [dv])
                            pend = gt < v

                            def retry(p):
                                plsc.store_scatter(accs[c], [dv], v, mask=p)
                                g2 = plsc.load_gather(accs[c], [dv])
                                return g2 < v

                            lax.while_loop(lambda p: jnp.any(p), retry, pend)
            return _
        lax.fori_loop(0, SK // (16 * NG), octgrp, 0)

    fire(0, 0)
    fire(1, 1)

    def pair(k, _):
        ci = k * 2
        process(ci, 0)

        @pl.when(ci + 2 < SN)
        def _f0():
            fire(ci + 2, 0)
        process(ci + 1, 1)

        @pl.when(ci + 3 < SN)
        def _f1():
            fire(ci + 3, 1)
        return _

    lax.fori_loop(0, SN // 2, pair, 0)
    if SN % 2:
        process(SN - 1, 0)
    for c in range(RPW):
        pltpu.sync_copy(accs[c], out_hbm.at[h, r0 + c, :])


BE = 2560


def _edge_body(xg_ref, ea_ref, wx_ref, we_ref, w1b_ref, out_ref):
    t1 = lax.dot_general(wx_ref[...], xg_ref[...], (((1,), (1,)), ((), ())),
                         preferred_element_type=jnp.float32)
    t2 = lax.dot_general(we_ref[...], ea_ref[...], (((1,), (1,)), ((), ())),
                         preferred_element_type=jnp.float32)
    h = jnp.maximum(t1 + t2, 0.0)
    m = lax.dot_general(w1b_ref[...], h, (((1,), (0,)), ((), ())),
                        preferred_element_type=jnp.float32)
    out_ref[...] = jnp.maximum(m, 0.0)


def _tc_edge(xg, ea, wx, we, w1b):
    return pl.pallas_call(
        _edge_body,
        grid=(E // BE,),
        in_specs=[
            pl.BlockSpec((BE, D), lambda i: (i, 0)),
            pl.BlockSpec((BE, DE), lambda i: (i, 0)),
            pl.BlockSpec((256, D), lambda i: (0, 0)),
            pl.BlockSpec((256, DE), lambda i: (0, 0)),
            pl.BlockSpec((D, 256), lambda i: (0, 0)),
        ],
        out_specs=pl.BlockSpec((D, BE), lambda i: (0, i)),
        out_shape=jax.ShapeDtypeStruct((D, E), jnp.float32),
    )(xg, ea, wx, we, w1b)


BN = 1024


def _node_body(x_ref, at_ref, wxa_ref, wag_ref, w2b_ref, out_ref):
    x = x_ref[...]
    at = jnp.maximum(at_ref[0], at_ref[1])
    t1 = lax.dot_general(x, wxa_ref[...], (((1,), (1,)), ((), ())),
                         preferred_element_type=jnp.float32)
    t2 = lax.dot_general(at, wag_ref[...], (((0,), (1,)), ((), ())),
                         preferred_element_type=jnp.float32)
    h = jnp.maximum(t1 + t2, 0.0)
    c = lax.dot_general(h, w2b_ref[...], (((1,), (1,)), ((), ())),
                        preferred_element_type=jnp.float32)
    c = jnp.maximum(c, 0.0)
    nor = jnp.sqrt(jnp.sum(c * c, axis=1, keepdims=True))
    c = c / jnp.maximum(1.0, nor)
    out_ref[...] = jnp.concatenate([c, x[:, : D // 2]], axis=1)


def _tc_node(xp, aggrT, wxa, wag, w2b):
    return pl.pallas_call(
        _node_body,
        grid=(NP // BN,),
        in_specs=[
            pl.BlockSpec((BN, D), lambda i: (i, 0)),
            pl.BlockSpec((2, D, BN), lambda i: (0, 0, i)),
            pl.BlockSpec((256, D), lambda i: (0, 0)),
            pl.BlockSpec((256, D), lambda i: (0, 0)),
            pl.BlockSpec((64, 256), lambda i: (0, 0)),
        ],
        out_specs=pl.BlockSpec((BN, D), lambda i: (i, 0)),
        out_shape=jax.ShapeDtypeStruct((NP, D), jnp.float32),
    )(xp, aggrT, wxa, wag, w2b)


def kernel(x, edge_index, edge_attr, w1a, w1b, w2a, w2b, if_bias):
    src = edge_index[0]
    dst = edge_index[1]
    wx, we = w1a[:, :D], w1a[:, D:]
    wxa, wag = w2a[:, :D], w2a[:, D:]
    xp = jnp.pad(x, ((0, NP - N), (0, 0)))
    for _ in range(3):
        xg = _sc_gather(xp, src)
        msgT = _tc_edge(xg, edge_attr, wx, we, w1b)
        aggrT = _sc_segmax(msgT, dst)
        xp = _tc_node(xp, aggrT, wxa, wag, w2b)
    return xp[:N]


# speedup vs baseline: 1.1953x; 1.1953x over previous
import functools

import jax
import jax.numpy as jnp
from jax import lax
from jax.experimental import pallas as pl
from jax.experimental.pallas import tpu as pltpu
from jax.experimental.pallas import tpu_sc as plsc

N = 10000
NP = 10240
E = 320000
D = 128
DE = 16
NW = 32

EPW = E // NW
GC = 400
GN = EPW // GC

_sc_mesh = plsc.VectorSubcoreMesh(core_axis_name="c", subcore_axis_name="s")


@functools.partial(
    pl.kernel,
    mesh=_sc_mesh,
    out_type=jax.ShapeDtypeStruct((E, D), jnp.float32),
    scratch_types=[
        pltpu.VMEM((EPW,), jnp.int32),
        pltpu.VMEM((GC, D), jnp.float32),
        pltpu.VMEM((GC, D), jnp.float32),
        pltpu.SemaphoreType.DMA,
        pltpu.SemaphoreType.DMA,
    ],
)
def _sc_gather(table_hbm, idx_hbm, out_hbm, idxa, rows0, rows1, sg0, sg1):
    wid = lax.axis_index("s") * 2 + lax.axis_index("c")
    base = pl.multiple_of(wid * EPW, 8)
    rows = (rows0, rows1)
    sgs = (sg0, sg1)

    pltpu.sync_copy(idx_hbm.at[pl.ds(base, EPW)], idxa)

    def fire(ci, b):
        o = pl.multiple_of(ci * GC, 8)
        pltpu.async_copy(table_hbm.at[idxa.at[pl.ds(o, GC)]], rows[b], sgs[b])

    def process(ci, b):
        o = pl.multiple_of(ci * GC, 8)
        pltpu.make_async_copy(table_hbm.at[idxa.at[pl.ds(o, GC)]],
                              rows[b], sgs[b]).wait()
        off = pl.multiple_of(base + ci * GC, 8)
        pltpu.sync_copy(rows[b], out_hbm.at[pl.ds(off, GC)])

    fire(0, 0)
    fire(1, 1)

    def pair(k, _):
        ci = k * 2
        process(ci, 0)

        @pl.when(ci + 2 < GN)
        def _f0():
            fire(ci + 2, 0)
        process(ci + 1, 1)

        @pl.when(ci + 3 < GN)
        def _f1():
            fire(ci + 3, 1)
        return _

    lax.fori_loop(0, GN // 2, pair, 0)
    if GN % 2:
        process(GN - 1, 0)


RPW = 8
EH = E // 2
SK = 1280
SN = EH // SK


@functools.partial(
    pl.kernel,
    mesh=_sc_mesh,
    out_type=jax.ShapeDtypeStruct((2, D, NP), jnp.float32),
    scratch_types=[
        [pltpu.VMEM((NP,), jnp.float32) for _ in range(RPW)],
        pltpu.VMEM((SK,), jnp.int32),
        pltpu.VMEM((SK,), jnp.int32),
        pltpu.VMEM((RPW, SK), jnp.float32),
        pltpu.VMEM((RPW, SK), jnp.float32),
        pltpu.VMEM((NP,), jnp.int32),
        pltpu.SemaphoreType.DMA,
        pltpu.SemaphoreType.DMA,
        pltpu.SemaphoreType.DMA,
        pltpu.SemaphoreType.DMA,
    ],
    compiler_params=pltpu.CompilerParams(needs_layout_passes=False),
)
def _sc_segmax(msgT_hbm, dst_hbm, out_hbm, accs, dstc0, dstc1, mv0, mv1,
               tag, sd0, sd1, sm0, sm1):
    wid = lax.axis_index("s") * 2 + lax.axis_index("c")
    g = wid % 16
    h = wid // 16
    r0 = pl.multiple_of(g * RPW, 8)
    eb = pl.multiple_of(h * EH, 128)
    zeros = jnp.zeros((16,), jnp.float32)
    dstcs = (dstc0, dstc1)
    mvs = (mv0, mv1)
    sds = (sd0, sd1)
    sms = (sm0, sm1)

    def zbody(j, _):
        for c in range(RPW):
            accs[c][pl.ds(j * 16, 16)] = zeros
        return _
    lax.fori_loop(0, NP // 16, zbody, 0)

    def _srcs(ci):
        e0 = pl.multiple_of(eb + ci * SK, 128)
        return (dst_hbm.at[pl.ds(e0, SK)],
                msgT_hbm.at[pl.ds(r0, RPW), pl.ds(e0, SK)])

    def fire(ci, b):
        sd, sm = _srcs(ci)
        pltpu.async_copy(sd, dstcs[b], sds[b])
        pltpu.async_copy(sm, mvs[b], sms[b])

    def process(ci, b):
        dstc, mv = dstcs[b], mvs[b]
        sd, sm = _srcs(ci)
        pltpu.make_async_copy(sd, dstc, sds[b]).wait()
        pltpu.make_async_copy(sm, mv, sms[b]).wait()

        lanes = lax.iota(jnp.int32, 16)

        NG = 4

        def quadgrp(jp, _):
            j0 = jp * NG
            dvs = [dstc[pl.ds((j0 + q) * 16, 16)] for q in range(NG)]
            anydup = None
            for dv in dvs:
                plsc.store_scatter(tag, [dv], lanes)
                rb = plsc.load_gather(tag, [dv])
                d = rb != lanes
                anydup = d if anydup is None else (anydup | d)
            hasdup = jnp.any(anydup)

            @pl.when(jnp.logical_not(hasdup))
            def _fast():
                for q, dv in enumerate(dvs):
                    for c in range(RPW):
                        v = mv[c, pl.ds((j0 + q) * 16, 16)]
                        a = plsc.load_gather(accs[c], [dv])
                        plsc.store_scatter(accs[c], [dv], jnp.maximum(a, v))

            @pl.when(hasdup)
            def _slow():
                for q, dv in enumerate(dvs):
                    pendor = None
                    for c in range(RPW):
                        v = mv[c, pl.ds((j0 + q) * 16, 16)]
                        a = plsc.load_gather(accs[c], [dv])
                        plsc.store_scatter(accs[c], [dv], jnp.maximum(a, v))
                        gt = plsc.load_gather(accs[c], [dv])
                        p = gt < v
                        pendor = p if pendor is None else (pendor | p)

                    @pl.when(jnp.any(pendor))
                    def _fix(q=q, dv=dv):
                        for c in range(RPW):
                            v = mv[c, pl.ds((j0 + q) * 16, 16)]
                            gt = plsc.load_gather(accs[c], [dv])
                            pend = gt < v

                            def retry(p):
                                plsc.store_scatter(accs[c], [dv], v, mask=p)
                                g2 = plsc.load_gather(accs[c], [dv])
                                return g2 < v

                            lax.while_loop(lambda p: jnp.any(p), retry, pend)
            return _
        lax.fori_loop(0, SK // (16 * NG), quadgrp, 0)

    fire(0, 0)
    fire(1, 1)

    def pair(k, _):
        ci = k * 2
        process(ci, 0)

        @pl.when(ci + 2 < SN)
        def _f0():
            fire(ci + 2, 0)
        process(ci + 1, 1)

        @pl.when(ci + 3 < SN)
        def _f1():
            fire(ci + 3, 1)
        return _

    lax.fori_loop(0, SN // 2, pair, 0)
    if SN % 2:
        process(SN - 1, 0)
    for c in range(RPW):
        pltpu.sync_copy(accs[c], out_hbm.at[h, r0 + c, :])


BE = 2560


def _edge_body(xg_ref, ea_ref, wx_ref, we_ref, w1b_ref, out_ref):
    t1 = lax.dot_general(wx_ref[...], xg_ref[...], (((1,), (1,)), ((), ())),
                         preferred_element_type=jnp.float32)
    t2 = lax.dot_general(we_ref[...], ea_ref[...], (((1,), (1,)), ((), ())),
                         preferred_element_type=jnp.float32)
    h = jnp.maximum(t1 + t2, 0.0)
    m = lax.dot_general(w1b_ref[...], h, (((1,), (0,)), ((), ())),
                        preferred_element_type=jnp.float32)
    out_ref[...] = jnp.maximum(m, 0.0)


def _tc_edge(xg, ea, wx, we, w1b):
    return pl.pallas_call(
        _edge_body,
        grid=(E // BE,),
        in_specs=[
            pl.BlockSpec((BE, D), lambda i: (i, 0)),
            pl.BlockSpec((BE, DE), lambda i: (i, 0)),
            pl.BlockSpec((256, D), lambda i: (0, 0)),
            pl.BlockSpec((256, DE), lambda i: (0, 0)),
            pl.BlockSpec((D, 256), lambda i: (0, 0)),
        ],
        out_specs=pl.BlockSpec((D, BE), lambda i: (0, i)),
        out_shape=jax.ShapeDtypeStruct((D, E), jnp.float32),
    )(xg, ea, wx, we, w1b)


BN = 1024


def _node_body(x_ref, at_ref, wxa_ref, wag_ref, w2b_ref, out_ref):
    x = x_ref[...]
    at = jnp.maximum(at_ref[0], at_ref[1])
    t1 = lax.dot_general(x, wxa_ref[...], (((1,), (1,)), ((), ())),
                         preferred_element_type=jnp.float32)
    t2 = lax.dot_general(at, wag_ref[...], (((0,), (1,)), ((), ())),
                         preferred_element_type=jnp.float32)
    h = jnp.maximum(t1 + t2, 0.0)
    c = lax.dot_general(h, w2b_ref[...], (((1,), (1,)), ((), ())),
                        preferred_element_type=jnp.float32)
    c = jnp.maximum(c, 0.0)
    nor = jnp.sqrt(jnp.sum(c * c, axis=1, keepdims=True))
    c = c / jnp.maximum(1.0, nor)
    out_ref[...] = jnp.concatenate([c, x[:, : D // 2]], axis=1)


def _tc_node(xp, aggrT, wxa, wag, w2b):
    return pl.pallas_call(
        _node_body,
        grid=(NP // BN,),
        in_specs=[
            pl.BlockSpec((BN, D), lambda i: (i, 0)),
            pl.BlockSpec((2, D, BN), lambda i: (0, 0, i)),
            pl.BlockSpec((256, D), lambda i: (0, 0)),
            pl.BlockSpec((256, D), lambda i: (0, 0)),
            pl.BlockSpec((64, 256), lambda i: (0, 0)),
        ],
        out_specs=pl.BlockSpec((BN, D), lambda i: (i, 0)),
        out_shape=jax.ShapeDtypeStruct((NP, D), jnp.float32),
    )(xp, aggrT, wxa, wag, w2b)


def kernel(x, edge_index, edge_attr, w1a, w1b, w2a, w2b, if_bias):
    src = edge_index[0]
    dst = edge_index[1]
    wx, we = w1a[:, :D], w1a[:, D:]
    wxa, wag = w2a[:, :D], w2a[:, D:]
    xp = jnp.pad(x, ((0, NP - N), (0, 0)))
    for _ in range(3):
        xg = _sc_gather(xp, src)
        msgT = _tc_edge(xg, edge_attr, wx, we, w1b)
        aggrT = _sc_segmax(msgT, dst)
        xp = _tc_node(xp, aggrT, wxa, wag, w2b)
    return xp[:N]
